# Initial kernel scaffold; baseline (speedup 1.0000x reference)
import jax
import jax.numpy as jnp
from jax.experimental import pallas as pl


def kernel(x, table):
    raise NotImplementedError("write your pallas kernel here")


# speedup vs baseline: 3.7670x; 3.7670x over previous
import jax
import jax.numpy as jnp
from jax.experimental import pallas as pl

SEQ_LEN = 200
EMBED_DIM = 64
B_BLK = 128


def _body(x_ref, pe_ref, o_ref):
    xb = x_ref[...]
    rowsum = jnp.sum(xb, axis=2, keepdims=True)
    pe = pe_ref[...]
    keep = (rowsum != 0.0).astype(xb.dtype)
    o_ref[...] = xb + keep * pe[None, :, :]


def kernel(x, table):
    B = x.shape[0]
    pe = table[1:, :]
    grid = (B // B_BLK,)
    return pl.pallas_call(
        _body,
        grid=grid,
        in_specs=[
            pl.BlockSpec((B_BLK, SEQ_LEN, EMBED_DIM), lambda i: (i, 0, 0)),
            pl.BlockSpec((SEQ_LEN, EMBED_DIM), lambda i: (0, 0)),
        ],
        out_specs=pl.BlockSpec((B_BLK, SEQ_LEN, EMBED_DIM), lambda i: (i, 0, 0)),
        out_shape=jax.ShapeDtypeStruct(x.shape, x.dtype),
    )(x, pe)
